# baseline (speedup 1.0000x reference)
import functools

import jax
import jax.numpy as jnp
from jax import lax
from jax.experimental import pallas as pl
from jax.experimental.pallas import tpu as pltpu
from jax.experimental.pallas import tpu_sc as plsc

NC, NS = 2, 16
K = 12
B = 16


def _chunks(total, cap):
    out, off = [], 0
    while off < total:
        s = min(cap, total - off)
        out.append((off, s))
        off += s
    return out


def _cheb_sc(N, NUPD, WH, EPT, BLK, CAPX, CAPU, PIPE):
    NB = EPT // BLK
    RPTX = N // NS
    CHX = _chunks(RPTX, CAPX)
    RPT = NUPD // NS
    CH = _chunks(RPT, CAPU)
    NV = WH // 16
    PADR = N - NUPD
    mesh = plsc.VectorSubcoreMesh(core_axis_name="c", subcore_axis_name="s")

    @functools.partial(
        pl.kernel,
        out_type=jax.ShapeDtypeStruct((2 * K, N, WH), jnp.float32),
        mesh=mesh,
        scratch_types=(
            [pltpu.VMEM((BLK, WH), jnp.float32)] * (2 if PIPE else 1)
            + [pltpu.VMEM((CAPX, WH), jnp.float32),
               pltpu.VMEM((CAPU, WH), jnp.float32)]
            + [pltpu.VMEM((2, BLK), jnp.int32)] * (2 if PIPE else 1)
            + [pltpu.VMEM((BLK,), jnp.float32)] * (2 if PIPE else 1)
            + [pltpu.VMEM((48,), jnp.int32),
               pltpu.VMEM((RPT, WH), jnp.float32)]
            + [pltpu.SemaphoreType.DMA] * (2 if PIPE else 1)
        ),
    )
    def cheb(xp, eH, wH, bbH, zrow, ts, *scr):
        if PIPE:
            (gbuf0, gbuf1, abuf, pbuf, ebuf0, ebuf1, wv0, wv1, bbv, acc,
             sem0, sem1) = scr
        else:
            gbuf0, abuf, pbuf, ebuf0, wv0, bbv, acc, sem0 = scr
        cid = lax.axis_index("c")
        sid = lax.axis_index("s")
        r0 = sid * RPT

        for co, RS in CHX:
            rr = sid * RPTX + co
            pltpu.sync_copy(xp.at[cid].at[pl.ds(rr, RS)],
                            abuf.at[pl.ds(0, RS)])
            pltpu.sync_copy(abuf.at[pl.ds(0, RS)],
                            ts.at[cid].at[pl.ds(rr, RS)])
        pltpu.sync_copy(zrow, acc)
        pltpu.sync_copy(bbH, bbv)
        if PADR:
            @pl.when(sid == 0)
            def _():
                pltpu.sync_copy(zrow.at[pl.ds(0, CAPX)], abuf)
                for k in range(1, K):
                    for po, ps in _chunks(PADR, CAPX):
                        pltpu.sync_copy(
                            abuf.at[pl.ds(0, ps)],
                            ts.at[2 * k + cid].at[pl.ds(NUPD + po, ps)])
        lo_b = bbv[pl.ds(sid, 16)][0]
        hi_b = bbv[pl.ds(sid + 16, 16)][0]
        plsc.subcore_barrier()

        nblk = hi_b - lo_b

        def mk_grp(gb, eb, wb):
            def grp(g, ___):
                sl16 = pl.ds(g * 16, 16)
                dvec = eb[1, sl16]
                wvec = wb[sl16]
                m = (dvec >= r0) & (dvec < r0 + RPT)
                wm = jnp.where(m, wvec, 0.0)
                dloc = jnp.where(m, dvec - r0, 0)
                for ei in range(16):
                    d = dloc[ei]
                    we = wm[ei]
                    e = g * 16 + ei
                    for j in range(NV):
                        sl = pl.ds(j * 16, 16)
                        plsc.addupdate(acc.at[d, sl], gb[e, sl] * we)
                return 0
            return grp

        def kbody(k, _):
            s_prev = 2 * (k - 1) + cid
            s_cur = 2 * k + cid

            if PIPE:
                BUFS = ((gbuf0, ebuf0, wv0, sem0), (gbuf1, ebuf1, wv1, sem1))

                @pl.when(nblk > 0)
                def _():
                    pltpu.sync_copy(eH.at[lo_b], ebuf0)
                    pltpu.sync_copy(wH.at[pl.ds(lo_b * BLK, BLK)], wv0)
                    pltpu.async_copy(ts.at[s_prev].at[ebuf0.at[0]], gbuf0,
                                     sem0)

                def blk_body(t, __):
                    bi = lo_b + t
                    for p in (0, 1):
                        gb, eb, wb, sm = BUFS[p]
                        gb2, eb2, wb2, sm2 = BUFS[1 - p]

                        @pl.when((t & 1) == p)
                        def _(gb=gb, eb=eb, wb=wb, sm=sm,
                              gb2=gb2, eb2=eb2, wb2=wb2, sm2=sm2):
                            @pl.when(t + 1 < nblk)
                            def _():
                                pltpu.sync_copy(eH.at[bi + 1], eb2)
                                pltpu.sync_copy(
                                    wH.at[pl.ds((bi + 1) * BLK, BLK)], wb2)
                                pltpu.async_copy(
                                    ts.at[s_prev].at[eb2.at[0]], gb2, sm2)

                            pltpu.make_async_copy(
                                ts.at[s_prev].at[eb.at[0]], gb, sm).wait()
                            lax.fori_loop(0, BLK // 16, mk_grp(gb, eb, wb),
                                          0, unroll=2)
                    return 0

                lax.fori_loop(0, nblk, blk_body, 0)
            else:
                def blk_body(bi, __):
                    pltpu.sync_copy(eH.at[bi], ebuf0)
                    pltpu.sync_copy(wH.at[pl.ds(bi * BLK, BLK)], wv0)
                    pltpu.async_copy(ts.at[s_prev].at[ebuf0.at[0]], gbuf0,
                                     sem0).wait()
                    lax.fori_loop(0, BLK // 16, mk_grp(gbuf0, ebuf0, wv0),
                                  0, unroll=2)
                    return 0

                lax.fori_loop(lo_b, hi_b, blk_body, 0)

            for co, RS in CH:
                @pl.when(k >= 2)
                def _(co=co, RS=RS):
                    pltpu.sync_copy(
                        ts.at[2 * (k - 2) + cid].at[pl.ds(r0 + co, RS)],
                        pbuf.at[pl.ds(0, RS)])

                    def upd(r, __, co=co):
                        for jj in range(NV):
                            sl = pl.ds(jj * 16, 16)
                            acc[co + r, sl] = (2.0 * acc[co + r, sl]
                                               - pbuf[r, sl])
                        return 0

                    lax.fori_loop(0, RS, upd, 0)

                pltpu.sync_copy(acc.at[pl.ds(co, RS)],
                                ts.at[s_cur].at[pl.ds(r0 + co, RS)])
            pltpu.sync_copy(zrow, acc)
            plsc.subcore_barrier()
            return 0

        lax.fori_loop(1, K, kbody, 0)

    return cheb


_cheb1 = _cheb_sc(N=10240, NUPD=10112, WH=128, EPT=161792, BLK=128,
                  CAPX=32, CAPU=64, PIPE=True)
_cheb2 = _cheb_sc(N=1024, NUPD=1024, WH=512, EPT=16384, BLK=128,
                  CAPX=24, CAPU=24, PIPE=False)


def _prep_edges(src, dst, w, ept, nupd, blk):
    e = src.shape[0]
    order = jnp.argsort(dst)
    src, dst, w = src[order], dst[order], w[order]
    p = ept - e
    src = jnp.concatenate([src, jnp.arange(p, dtype=jnp.int32) % nupd])
    dst = jnp.concatenate([dst, jnp.full((p,), nupd - 1, jnp.int32)])
    w = jnp.concatenate([w, jnp.zeros((p,), jnp.float32)])
    rpt = nupd // NS
    bounds = jnp.arange(NS + 1, dtype=jnp.int32) * rpt
    ptr = jnp.searchsorted(dst, bounds).astype(jnp.int32)
    bs = ptr[:NS] // blk
    be = (ptr[1:] + blk - 1) // blk
    bb = jnp.concatenate([bs, be, jnp.zeros((16,), jnp.int32)])
    nbt = ept // blk
    packed = jnp.stack([src.reshape(nbt, blk), dst.reshape(nbt, blk)],
                       axis=1)
    return packed, w, bb


def _delta_expand(We, cols):
    Kk, H, F = We.shape
    eye = jnp.eye(B, dtype=jnp.float32)
    M = We[:, :, None, :, None] * eye[None, None, :, None, :]
    M = M.reshape(Kk, H * B, F * B)
    M = jnp.pad(M, ((0, 0), (0, cols - H * B), (0, 0)))
    return M.reshape(Kk, 2, cols // 2, F * B).reshape(2 * Kk, cols // 2, F * B)


def _stack_matmul(ts, M, bias_row, n_rows, n_blk, relu):
    S = ts.shape[0]
    bn = n_rows // n_blk
    fin, fout = ts.shape[2], M.shape[2]
    last = S - 1

    def body(ts_ref, m_ref, b_ref, o_ref):
        s = pl.program_id(1)
        p = jnp.dot(ts_ref[0], m_ref[0], preferred_element_type=jnp.float32)

        @pl.when(s == 0)
        def _():
            o_ref[...] = p

        @pl.when(s > 0)
        def _():
            o_ref[...] += p

        @pl.when(s == last)
        def _():
            r = o_ref[...] + b_ref[...]
            o_ref[...] = jnp.maximum(r, 0.0) if relu else r

    return pl.pallas_call(
        body,
        grid=(n_blk, S),
        in_specs=[
            pl.BlockSpec((1, bn, fin), lambda n, s: (s, n, 0)),
            pl.BlockSpec((1, fin, fout), lambda n, s: (s, 0, 0)),
            pl.BlockSpec((1, fout), lambda n, s: (0, 0)),
        ],
        out_specs=pl.BlockSpec((bn, fout), lambda n, s: (n, 0)),
        out_shape=jax.ShapeDtypeStruct((n_rows, fout), jnp.float32),
    )(ts, M, bias_row)


def _coarsen(bmat, h1):
    n2, n1 = bmat.shape
    f = h1.shape[1]
    kb = 8
    bk = n1 // kb

    def body(b_ref, h_ref, o_ref):
        k = pl.program_id(0)
        p = jnp.dot(b_ref[...], h_ref[...], preferred_element_type=jnp.float32)

        @pl.when(k == 0)
        def _():
            o_ref[...] = p

        @pl.when(k > 0)
        def _():
            o_ref[...] += p

    return pl.pallas_call(
        body,
        grid=(kb,),
        in_specs=[
            pl.BlockSpec((n2, bk), lambda k: (0, k)),
            pl.BlockSpec((bk, f), lambda k: (k, 0)),
        ],
        out_specs=pl.BlockSpec((n2, f), lambda k: (0, 0)),
        out_shape=jax.ShapeDtypeStruct((n2, f), jnp.float32),
    )(bmat, h1)


def _fc_logsoftmax(a, Wfc, bfc):
    def body(a_ref, w_ref, b_ref, o_ref):
        logits = jnp.dot(a_ref[...], w_ref[...],
                         preferred_element_type=jnp.float32) + b_ref[...]
        m = jnp.max(logits, axis=1, keepdims=True)
        e = jnp.exp(logits - m)
        o_ref[...] = (logits - m) - jnp.log(jnp.sum(e, axis=1, keepdims=True))

    nb, nc = a.shape[0], Wfc.shape[1]
    return pl.pallas_call(
        body,
        out_shape=jax.ShapeDtypeStruct((nb, nc), jnp.float32),
    )(a, Wfc, bfc.reshape(1, nc))


def kernel(x, edge_index1, edge_weight1, edge_index2, edge_weight2, b, W1, b1,
           W2, b2, Wfc, bfc):
    Bq, N1, T = x.shape
    N2 = b.shape[0]
    N1P, N2P = 10240, 1024

    xp = jnp.transpose(x, (1, 2, 0)).reshape(N1, T * B)
    xp = jnp.pad(xp, ((0, N1P - N1), (0, 256 - T * B)))
    xp1 = xp.reshape(N1P, 2, 128).transpose(1, 0, 2)
    e1, w1, bb1 = _prep_edges(
        edge_index1[0], edge_index1[1], edge_weight1, 161792, 10112, 128)
    zeros1 = jnp.zeros((10112 // NS, 128), jnp.float32)

    ts1 = _cheb1(xp1, e1, w1, bb1, zeros1)

    M1 = _delta_expand(W1[:, :, 0, :], 256)
    bias1 = jnp.repeat(b1, B).reshape(1, 64 * B)
    h1 = _stack_matmul(ts1, M1, bias1, N1P, 10, relu=True)

    bp = jnp.pad(b, ((0, 0), (0, N1P - N1)))
    h2 = _coarsen(bp, h1)

    h2p = jnp.pad(h2, ((0, N2P - N2), (0, 0)))
    xp2 = h2p.reshape(N2P, 2, 512).transpose(1, 0, 2)
    e2, w2, bb2 = _prep_edges(
        edge_index2[0], edge_index2[1], edge_weight2, 16384, N2P, 128)
    zeros2 = jnp.zeros((N2P // NS, 512), jnp.float32)
    ts2 = _cheb2(xp2, e2, w2, bb2, zeros2)

    M2 = _delta_expand(W2[:, 0, :, :], 1024)
    bias2 = jnp.repeat(b2, B).reshape(1, 32 * B)
    hc = _stack_matmul(ts2, M2, bias2, N2P, 1, relu=False)

    a = hc[:N2].reshape(B, N2 * 32 * B // B)
    return _fc_logsoftmax(a, Wfc, bfc)


# speedup vs baseline: 1.0058x; 1.0058x over previous
import functools

import jax
import jax.numpy as jnp
from jax import lax
from jax.experimental import pallas as pl
from jax.experimental.pallas import tpu as pltpu
from jax.experimental.pallas import tpu_sc as plsc

NC, NS = 2, 16
K = 12
B = 16


def _chunks(total, cap):
    out, off = [], 0
    while off < total:
        s = min(cap, total - off)
        out.append((off, s))
        off += s
    return out


def _cheb_sc(N, NUPD, WH, EPT, BLK, CAPX, CAPU):
    NB = EPT // BLK
    RPTX = N // NS
    CHX = _chunks(RPTX, CAPX)
    RPT = NUPD // NS
    CH = _chunks(RPT, CAPU)
    NV = WH // 16
    PADR = N - NUPD
    mesh = plsc.VectorSubcoreMesh(core_axis_name="c", subcore_axis_name="s")

    @functools.partial(
        pl.kernel,
        out_type=jax.ShapeDtypeStruct((2 * K, N, WH), jnp.float32),
        mesh=mesh,
        scratch_types=[
            pltpu.VMEM((BLK, WH), jnp.float32),
            pltpu.VMEM((BLK, WH), jnp.float32),
            pltpu.VMEM((CAPX, WH), jnp.float32),
            pltpu.VMEM((CAPU, WH), jnp.float32),
            pltpu.VMEM((2, BLK), jnp.int32),
            pltpu.VMEM((2, BLK), jnp.int32),
            pltpu.VMEM((BLK,), jnp.float32),
            pltpu.VMEM((BLK,), jnp.float32),
            pltpu.VMEM((48,), jnp.int32),
            pltpu.VMEM((RPT, WH), jnp.float32),
            pltpu.SemaphoreType.DMA,
            pltpu.SemaphoreType.DMA,
        ],
    )
    def cheb(xp, eH, wH, bbH, zrow, ts,
             gbuf0, gbuf1, abuf, pbuf, ebuf0, ebuf1, wv0, wv1, bbv, acc,
             sem0, sem1):
        cid = lax.axis_index("c")
        sid = lax.axis_index("s")
        r0 = sid * RPT

        for co, RS in CHX:
            rr = sid * RPTX + co
            pltpu.sync_copy(xp.at[cid].at[pl.ds(rr, RS)],
                            abuf.at[pl.ds(0, RS)])
            pltpu.sync_copy(abuf.at[pl.ds(0, RS)],
                            ts.at[cid].at[pl.ds(rr, RS)])
        pltpu.sync_copy(zrow, acc)
        pltpu.sync_copy(bbH, bbv)
        if PADR:
            @pl.when(sid == 0)
            def _():
                pltpu.sync_copy(zrow.at[pl.ds(0, CAPX)], abuf)
                for k in range(1, K):
                    for po, ps in _chunks(PADR, CAPX):
                        pltpu.sync_copy(
                            abuf.at[pl.ds(0, ps)],
                            ts.at[2 * k + cid].at[pl.ds(NUPD + po, ps)])
        lo_b = bbv[pl.ds(sid, 16)][0]
        hi_b = bbv[pl.ds(sid + 16, 16)][0]
        plsc.subcore_barrier()

        nblk = hi_b - lo_b
        BUFS = ((gbuf0, ebuf0, wv0, sem0), (gbuf1, ebuf1, wv1, sem1))

        def kbody(k, _):
            s_prev = 2 * (k - 1) + cid
            s_cur = 2 * k + cid

            @pl.when(nblk > 0)
            def _():
                pltpu.sync_copy(eH.at[lo_b], ebuf0)
                pltpu.sync_copy(wH.at[pl.ds(lo_b * BLK, BLK)], wv0)
                pltpu.async_copy(ts.at[s_prev].at[ebuf0.at[0]], gbuf0, sem0)

            def blk_body(t, __):
                bi = lo_b + t
                for p in (0, 1):
                    gb, eb, wb, sm = BUFS[p]
                    gb2, eb2, wb2, sm2 = BUFS[1 - p]

                    @pl.when((t & 1) == p)
                    def _(gb=gb, eb=eb, wb=wb, sm=sm,
                          gb2=gb2, eb2=eb2, wb2=wb2, sm2=sm2):
                        @pl.when(t + 1 < nblk)
                        def _():
                            pltpu.sync_copy(eH.at[bi + 1], eb2)
                            pltpu.sync_copy(
                                wH.at[pl.ds((bi + 1) * BLK, BLK)], wb2)
                            pltpu.async_copy(
                                ts.at[s_prev].at[eb2.at[0]], gb2, sm2)

                        pltpu.make_async_copy(
                            ts.at[s_prev].at[eb.at[0]], gb, sm).wait()

                        def grp(g, ___, gb=gb, eb=eb, wb=wb):
                            sl16 = pl.ds(g * 16, 16)
                            dvec = eb[1, sl16]
                            wvec = wb[sl16]
                            m = (dvec >= r0) & (dvec < r0 + RPT)
                            wm = jnp.where(m, wvec, 0.0)
                            dloc = jnp.where(m, dvec - r0, 0)
                            for ei in range(16):
                                d = dloc[ei]
                                we = wm[ei]
                                e = g * 16 + ei
                                for j in range(NV):
                                    sl = pl.ds(j * 16, 16)
                                    plsc.addupdate(acc.at[d, sl],
                                                   gb[e, sl] * we)
                            return 0

                        lax.fori_loop(0, BLK // 16, grp, 0)
                return 0

            lax.fori_loop(0, nblk, blk_body, 0)

            for co, RS in CH:
                @pl.when(k >= 2)
                def _(co=co, RS=RS):
                    pltpu.sync_copy(
                        ts.at[2 * (k - 2) + cid].at[pl.ds(r0 + co, RS)],
                        pbuf.at[pl.ds(0, RS)])

                    def upd(r, __, co=co):
                        for jj in range(NV):
                            sl = pl.ds(jj * 16, 16)
                            acc[co + r, sl] = (2.0 * acc[co + r, sl]
                                               - pbuf[r, sl])
                        return 0

                    lax.fori_loop(0, RS, upd, 0)

                pltpu.sync_copy(acc.at[pl.ds(co, RS)],
                                ts.at[s_cur].at[pl.ds(r0 + co, RS)])
            pltpu.sync_copy(zrow, acc)
            plsc.subcore_barrier()
            return 0

        lax.fori_loop(1, K, kbody, 0)

    return cheb


_cheb1 = _cheb_sc(N=10240, NUPD=10112, WH=128, EPT=161792, BLK=128,
                  CAPX=32, CAPU=64)
_cheb2 = _cheb_sc(N=1024, NUPD=1024, WH=512, EPT=16384, BLK=64,
                  CAPX=24, CAPU=24)


def _prep_edges(src, dst, w, ept, nupd, blk):
    e = src.shape[0]
    order = jnp.argsort(dst)
    src, dst, w = src[order], dst[order], w[order]
    p = ept - e
    src = jnp.concatenate([src, jnp.arange(p, dtype=jnp.int32) % nupd])
    dst = jnp.concatenate([dst, jnp.full((p,), nupd - 1, jnp.int32)])
    w = jnp.concatenate([w, jnp.zeros((p,), jnp.float32)])
    rpt = nupd // NS
    bounds = jnp.arange(NS + 1, dtype=jnp.int32) * rpt
    ptr = jnp.searchsorted(dst, bounds).astype(jnp.int32)
    bs = ptr[:NS] // blk
    be = (ptr[1:] + blk - 1) // blk
    bb = jnp.concatenate([bs, be, jnp.zeros((16,), jnp.int32)])
    nbt = ept // blk
    packed = jnp.stack([src.reshape(nbt, blk), dst.reshape(nbt, blk)],
                       axis=1)
    return packed, w, bb


def _delta_expand(We, cols):
    Kk, H, F = We.shape
    eye = jnp.eye(B, dtype=jnp.float32)
    M = We[:, :, None, :, None] * eye[None, None, :, None, :]
    M = M.reshape(Kk, H * B, F * B)
    M = jnp.pad(M, ((0, 0), (0, cols - H * B), (0, 0)))
    return M.reshape(Kk, 2, cols // 2, F * B).reshape(2 * Kk, cols // 2, F * B)


def _stack_matmul(ts, M, bias_row, n_rows, n_blk, relu):
    S = ts.shape[0]
    bn = n_rows // n_blk
    fin, fout = ts.shape[2], M.shape[2]
    last = S - 1

    def body(ts_ref, m_ref, b_ref, o_ref):
        s = pl.program_id(1)
        p = jnp.dot(ts_ref[0], m_ref[0], preferred_element_type=jnp.float32)

        @pl.when(s == 0)
        def _():
            o_ref[...] = p

        @pl.when(s > 0)
        def _():
            o_ref[...] += p

        @pl.when(s == last)
        def _():
            r = o_ref[...] + b_ref[...]
            o_ref[...] = jnp.maximum(r, 0.0) if relu else r

    return pl.pallas_call(
        body,
        grid=(n_blk, S),
        in_specs=[
            pl.BlockSpec((1, bn, fin), lambda n, s: (s, n, 0)),
            pl.BlockSpec((1, fin, fout), lambda n, s: (s, 0, 0)),
            pl.BlockSpec((1, fout), lambda n, s: (0, 0)),
        ],
        out_specs=pl.BlockSpec((bn, fout), lambda n, s: (n, 0)),
        out_shape=jax.ShapeDtypeStruct((n_rows, fout), jnp.float32),
    )(ts, M, bias_row)


def _coarsen(bmat, h1):
    n2, n1 = bmat.shape
    f = h1.shape[1]
    kb = 8
    bk = n1 // kb

    def body(b_ref, h_ref, o_ref):
        k = pl.program_id(0)
        p = jnp.dot(b_ref[...], h_ref[...], preferred_element_type=jnp.float32)

        @pl.when(k == 0)
        def _():
            o_ref[...] = p

        @pl.when(k > 0)
        def _():
            o_ref[...] += p

    return pl.pallas_call(
        body,
        grid=(kb,),
        in_specs=[
            pl.BlockSpec((n2, bk), lambda k: (0, k)),
            pl.BlockSpec((bk, f), lambda k: (k, 0)),
        ],
        out_specs=pl.BlockSpec((n2, f), lambda k: (0, 0)),
        out_shape=jax.ShapeDtypeStruct((n2, f), jnp.float32),
    )(bmat, h1)


def _fc_logsoftmax(a, Wfc, bfc):
    def body(a_ref, w_ref, b_ref, o_ref):
        logits = jnp.dot(a_ref[...], w_ref[...],
                         preferred_element_type=jnp.float32) + b_ref[...]
        m = jnp.max(logits, axis=1, keepdims=True)
        e = jnp.exp(logits - m)
        o_ref[...] = (logits - m) - jnp.log(jnp.sum(e, axis=1, keepdims=True))

    nb, nc = a.shape[0], Wfc.shape[1]
    return pl.pallas_call(
        body,
        out_shape=jax.ShapeDtypeStruct((nb, nc), jnp.float32),
    )(a, Wfc, bfc.reshape(1, nc))


def kernel(x, edge_index1, edge_weight1, edge_index2, edge_weight2, b, W1, b1,
           W2, b2, Wfc, bfc):
    Bq, N1, T = x.shape
    N2 = b.shape[0]
    N1P, N2P = 10240, 1024

    xp = jnp.transpose(x, (1, 2, 0)).reshape(N1, T * B)
    xp = jnp.pad(xp, ((0, N1P - N1), (0, 256 - T * B)))
    xp1 = xp.reshape(N1P, 2, 128).transpose(1, 0, 2)
    e1, w1, bb1 = _prep_edges(
        edge_index1[0], edge_index1[1], edge_weight1, 161792, 10112, 128)
    zeros1 = jnp.zeros((10112 // NS, 128), jnp.float32)

    ts1 = _cheb1(xp1, e1, w1, bb1, zeros1)

    M1 = _delta_expand(W1[:, :, 0, :], 256)
    bias1 = jnp.repeat(b1, B).reshape(1, 64 * B)
    h1 = _stack_matmul(ts1, M1, bias1, N1P, 10, relu=True)

    bp = jnp.pad(b, ((0, 0), (0, N1P - N1)))
    h2 = _coarsen(bp, h1)

    h2p = jnp.pad(h2, ((0, N2P - N2), (0, 0)))
    xp2 = h2p.reshape(N2P, 2, 512).transpose(1, 0, 2)
    e2, w2, bb2 = _prep_edges(
        edge_index2[0], edge_index2[1], edge_weight2, 16384, N2P, 64)
    zeros2 = jnp.zeros((N2P // NS, 512), jnp.float32)
    ts2 = _cheb2(xp2, e2, w2, bb2, zeros2)

    M2 = _delta_expand(W2[:, 0, :, :], 1024)
    bias2 = jnp.repeat(b2, B).reshape(1, 32 * B)
    hc = _stack_matmul(ts2, M2, bias2, N2P, 1, relu=False)

    a = hc[:N2].reshape(B, N2 * 32 * B // B)
    return _fc_logsoftmax(a, Wfc, bfc)
